# Initial kernel scaffold; baseline (speedup 1.0000x reference)
import jax
import jax.numpy as jnp
from jax.experimental import pallas as pl


def kernel(x, table):
    raise NotImplementedError("write your pallas kernel here")


# speedup vs baseline: 2.4145x; 2.4145x over previous
import functools
import math

import jax
import jax.numpy as jnp
from jax import lax
from jax.experimental import pallas as pl
from jax.experimental.pallas import tpu as pltpu
from jax.experimental.pallas import tpu_sc as plsc

VOCAB_ = 100000
D = 128
NC = 2
NS = 16
NW = NC * NS
B_TOTAL = 4096 * 50
PER_W = B_TOTAL // NW
CHUNK = 128
N_CHUNKS = PER_W // CHUNK
SCALE = math.sqrt(128.0)
L = 16


def _emb_body(table_hbm, idx_hbm, out_hbm, idx_v, rows_v, gsem):
    c = lax.axis_index("c")
    s = lax.axis_index("s")
    wid = s * NC + c
    pltpu.sync_copy(idx_hbm.at[wid], idx_v)
    out_base = wid * PER_W

    @pl.loop(0, N_CHUNKS)
    def _chunk(j):
        pltpu.async_copy(table_hbm.at[idx_v.at[j]], rows_v, gsem).wait()

        @pl.loop(0, CHUNK, unroll=2)
        def _row(r):
            for seg in range(D // L):
                sl = pl.ds(seg * L, L)
                rows_v[r, sl] = rows_v[r, sl] * SCALE

        pltpu.sync_copy(rows_v, out_hbm.at[pl.ds(out_base + j * CHUNK, CHUNK)])


@functools.partial(jax.jit, static_argnames=())
def _emb_call(idx2d, table):
    mesh = plsc.VectorSubcoreMesh(core_axis_name="c", subcore_axis_name="s")
    k = pl.kernel(
        _emb_body,
        out_type=jax.ShapeDtypeStruct((B_TOTAL, D), jnp.float32),
        mesh=mesh,
        scratch_types=[
            pltpu.VMEM((N_CHUNKS, CHUNK), jnp.int32),
            pltpu.VMEM((CHUNK, D), jnp.float32),
            pltpu.SemaphoreType.DMA,
        ],
    )
    return k(table, idx2d)


def kernel(x, table):
    bs, sl = x.shape
    idx2d = x.astype(jnp.int32).reshape(NW, N_CHUNKS, CHUNK)
    out = _emb_call(idx2d, table)
    return out.reshape(bs, sl, D)


# speedup vs baseline: 2.9135x; 1.2067x over previous
import functools
import math

import jax
import jax.numpy as jnp
from jax import lax
from jax.experimental import pallas as pl
from jax.experimental.pallas import tpu as pltpu
from jax.experimental.pallas import tpu_sc as plsc

VOCAB_ = 100000
D = 128
NC = 2
NS = 16
NW = NC * NS
B_TOTAL = 4096 * 50
PER_W = B_TOTAL // NW
CHUNK = 128
N_CHUNKS = PER_W // CHUNK
SCALE = math.sqrt(128.0)
L = 16


NBUF = 5


def _scale_block(buf):
    @pl.loop(0, CHUNK, unroll=4)
    def _row(r):
        for seg in range(D // L):
            sl = pl.ds(seg * L, L)
            buf[r, sl] = buf[r, sl] * SCALE


def _emb_body(table_hbm, idx_hbm, out_hbm, idx_v,
              b0, b1, b2, b3, b4, gsems, ssems):
    c = lax.axis_index("c")
    s = lax.axis_index("s")
    wid = s * NC + c
    bufs = (b0, b1, b2, b3, b4)
    pltpu.sync_copy(idx_hbm.at[wid], idx_v)
    out_base = wid * PER_W

    for b in range(NBUF - 1):
        pltpu.async_copy(table_hbm.at[idx_v.at[b]], bufs[b], gsems.at[b])

    @pl.loop(0, N_CHUNKS, step=NBUF)
    def _pass(j):
        for b in range(NBUF):
            ch = j + b
            nb = (b + NBUF - 1) % NBUF

            @pl.when(ch >= 1)
            def _drain():
                pltpu.make_async_copy(
                    bufs[nb], out_hbm.at[pl.ds(0, CHUNK)], ssems.at[nb]
                ).wait()

            @pl.when(ch + NBUF - 1 < N_CHUNKS)
            def _lookahead():
                pltpu.async_copy(
                    table_hbm.at[idx_v.at[ch + NBUF - 1]], bufs[nb],
                    gsems.at[nb])

            pltpu.make_async_copy(
                table_hbm.at[idx_v.at[ch]], bufs[b], gsems.at[b]).wait()
            _scale_block(bufs[b])
            pltpu.async_copy(
                bufs[b], out_hbm.at[pl.ds(out_base + ch * CHUNK, CHUNK)],
                ssems.at[b])

    lb = (N_CHUNKS - 1) % NBUF
    pltpu.make_async_copy(
        bufs[lb], out_hbm.at[pl.ds(0, CHUNK)], ssems.at[lb]).wait()


@functools.partial(jax.jit, static_argnames=())
def _emb_call(idx2d, table):
    mesh = plsc.VectorSubcoreMesh(core_axis_name="c", subcore_axis_name="s")
    k = pl.kernel(
        _emb_body,
        out_type=jax.ShapeDtypeStruct((B_TOTAL, D), jnp.float32),
        mesh=mesh,
        scratch_types=(
            [pltpu.VMEM((N_CHUNKS, CHUNK), jnp.int32)]
            + [pltpu.VMEM((CHUNK, D), jnp.float32) for _ in range(NBUF)]
            + [pltpu.SemaphoreType.DMA((NBUF,)),
               pltpu.SemaphoreType.DMA((NBUF,))]
        ),
    )
    return k(table, idx2d)


def kernel(x, table):
    bs, sl = x.shape
    idx2d = x.astype(jnp.int32).reshape(NW, N_CHUNKS, CHUNK)
    out = _emb_call(idx2d, table)
    return out.reshape(bs, sl, D)


# speedup vs baseline: 5.1792x; 1.7776x over previous
import functools
import math

import jax
import jax.numpy as jnp
from jax import lax
from jax.experimental import pallas as pl
from jax.experimental.pallas import tpu as pltpu
from jax.experimental.pallas import tpu_sc as plsc

BATCH = 4096
SEQ = 50
D = 128
NC = 2
NS = 16
NW = NC * NS
ROWS_PER_W = BATCH // NW
SCALE = math.sqrt(128.0)
L = 16
NBUF = 4


def _scale_block(buf):
    @pl.loop(0, SEQ, unroll=2)
    def _row(r):
        for seg in range(D // L):
            sl = pl.ds(seg * L, L)
            buf[r, sl] = buf[r, sl] * SCALE


def _emb_body(table_hbm, idx_hbm, out_hbm, idx_v,
              b0, b1, b2, b3, gsems, ssems):
    c = lax.axis_index("c")
    s = lax.axis_index("s")
    wid = s * NC + c
    bufs = (b0, b1, b2, b3)
    pltpu.sync_copy(idx_hbm.at[wid], idx_v)
    row_base = wid * ROWS_PER_W

    for b in range(NBUF - 1):
        pltpu.async_copy(table_hbm.at[idx_v.at[b]], bufs[b], gsems.at[b])

    @pl.loop(0, ROWS_PER_W, step=NBUF)
    def _pass(j):
        for b in range(NBUF):
            r = j + b
            nb = (b + NBUF - 1) % NBUF

            @pl.when(r >= 1)
            def _drain():
                pltpu.make_async_copy(
                    bufs[nb], out_hbm.at[row_base], ssems.at[nb]).wait()

            @pl.when(r + NBUF - 1 < ROWS_PER_W)
            def _lookahead():
                pltpu.async_copy(
                    table_hbm.at[idx_v.at[r + NBUF - 1]], bufs[nb],
                    gsems.at[nb])

            pltpu.make_async_copy(
                table_hbm.at[idx_v.at[r]], bufs[b], gsems.at[b]).wait()
            _scale_block(bufs[b])
            pltpu.async_copy(
                bufs[b], out_hbm.at[row_base + r], ssems.at[b])

    lb = (ROWS_PER_W - 1) % NBUF
    pltpu.make_async_copy(
        bufs[lb], out_hbm.at[row_base], ssems.at[lb]).wait()


@functools.partial(jax.jit, static_argnames=())
def _emb_call(idx3d, table):
    mesh = plsc.VectorSubcoreMesh(core_axis_name="c", subcore_axis_name="s")
    k = pl.kernel(
        _emb_body,
        out_type=jax.ShapeDtypeStruct((BATCH, SEQ, D), jnp.float32),
        mesh=mesh,
        scratch_types=(
            [pltpu.VMEM((ROWS_PER_W, SEQ), jnp.int32)]
            + [pltpu.VMEM((SEQ, D), jnp.float32) for _ in range(NBUF)]
            + [pltpu.SemaphoreType.DMA((NBUF,)),
               pltpu.SemaphoreType.DMA((NBUF,))]
        ),
    )
    return k(table, idx3d)


def kernel(x, table):
    idx3d = x.astype(jnp.int32).reshape(NW, ROWS_PER_W, SEQ)
    return _emb_call(idx3d, table)


# speedup vs baseline: 9.1945x; 1.7753x over previous
import functools
import math

import jax
import jax.numpy as jnp
from jax import lax
from jax.experimental import pallas as pl
from jax.experimental.pallas import tpu as pltpu
from jax.experimental.pallas import tpu_sc as plsc

BATCH = 4096
SEQ = 50
D = 128
NC = 2
NS = 16
NW = NC * NS
COLS_PER_W = BATCH // NW
SCALE = math.sqrt(128.0)
L = 16
NBUF = 5


def _scale_block(buf):
    @pl.loop(0, COLS_PER_W, unroll=4)
    def _row(r):
        for seg in range(D // L):
            sl = pl.ds(seg * L, L)
            buf[r, sl] = buf[r, sl] * SCALE


def _emb_body(table_hbm, idx_hbm, out_hbm, idx_v,
              b0, b1, b2, b3, b4, gsems, ssems):
    c = lax.axis_index("c")
    s = lax.axis_index("s")
    wid = s * NC + c
    bufs = (b0, b1, b2, b3, b4)
    col0 = wid * COLS_PER_W
    pltpu.sync_copy(idx_hbm.at[:, pl.ds(col0, COLS_PER_W)], idx_v)

    for b in range(NBUF - 1):
        pltpu.async_copy(table_hbm.at[idx_v.at[b]], bufs[b], gsems.at[b])

    @pl.loop(0, SEQ, step=NBUF)
    def _pass(j):
        for b in range(NBUF):
            t = j + b
            nb = (b + NBUF - 1) % NBUF

            @pl.when(t >= 1)
            def _drain():
                pltpu.make_async_copy(
                    bufs[nb], out_hbm.at[0, pl.ds(col0, COLS_PER_W)],
                    ssems.at[nb]).wait()

            @pl.when(t + NBUF - 1 < SEQ)
            def _lookahead():
                pltpu.async_copy(
                    table_hbm.at[idx_v.at[t + NBUF - 1]], bufs[nb],
                    gsems.at[nb])

            pltpu.make_async_copy(
                table_hbm.at[idx_v.at[t]], bufs[b], gsems.at[b]).wait()
            _scale_block(bufs[b])
            pltpu.async_copy(
                bufs[b], out_hbm.at[t, pl.ds(col0, COLS_PER_W)], ssems.at[b])

    lb = (SEQ - 1) % NBUF
    pltpu.make_async_copy(
        bufs[lb], out_hbm.at[0, pl.ds(col0, COLS_PER_W)], ssems.at[lb]).wait()


@functools.partial(jax.jit, static_argnames=())
def _emb_call(idx_t, table):
    mesh = plsc.VectorSubcoreMesh(core_axis_name="c", subcore_axis_name="s")
    k = pl.kernel(
        _emb_body,
        out_type=jax.ShapeDtypeStruct((SEQ, BATCH, D), jnp.float32),
        mesh=mesh,
        scratch_types=(
            [pltpu.VMEM((SEQ, COLS_PER_W), jnp.int32)]
            + [pltpu.VMEM((COLS_PER_W, D), jnp.float32) for _ in range(NBUF)]
            + [pltpu.SemaphoreType.DMA((NBUF,)),
               pltpu.SemaphoreType.DMA((NBUF,))]
        ),
    )
    return k(table, idx_t)


def kernel(x, table):
    idx_t = x.astype(jnp.int32).T
    out_t = _emb_call(idx_t, table)
    return out_t.transpose(1, 0, 2)


# speedup vs baseline: 9.4185x; 1.0244x over previous
import functools
import math

import jax
import jax.numpy as jnp
from jax import lax
from jax.experimental import pallas as pl
from jax.experimental.pallas import tpu as pltpu
from jax.experimental.pallas import tpu_sc as plsc

BATCH = 4096
SEQ = 50
D = 128
NC = 2
NS = 16
NW = NC * NS
COLS_PER_W = BATCH // NW
SCALE = math.sqrt(128.0)
L = 16
NBUF = 5


def _scale_block(buf):
    @pl.loop(0, COLS_PER_W, unroll=4)
    def _row(r):
        for seg in range(D // L):
            sl = pl.ds(seg * L, L)
            buf[r, sl] = buf[r, sl] * SCALE


def _emb_body(table_hbm, idx_hbm, out_hbm, idx_v,
              b0, b1, b2, b3, b4, gsems, ssems):
    c = lax.axis_index("c")
    s = lax.axis_index("s")
    wid = s * NC + c
    bufs = (b0, b1, b2, b3, b4)
    col0 = wid * COLS_PER_W
    pltpu.sync_copy(idx_hbm.at[:, pl.ds(col0, COLS_PER_W)], idx_v)

    for b in range(NBUF - 2):
        pltpu.async_copy(table_hbm.at[idx_v.at[b]], bufs[b], gsems.at[b])

    @pl.loop(0, SEQ, step=NBUF)
    def _pass(j):
        for b in range(NBUF):
            t = j + b
            nb = (b + NBUF - 2) % NBUF

            @pl.when(t >= 2)
            def _drain():
                pltpu.make_async_copy(
                    bufs[nb], out_hbm.at[0, pl.ds(col0, COLS_PER_W)],
                    ssems.at[nb]).wait()

            @pl.when(t + NBUF - 2 < SEQ)
            def _lookahead():
                pltpu.async_copy(
                    table_hbm.at[idx_v.at[t + NBUF - 2]], bufs[nb],
                    gsems.at[nb])

            pltpu.make_async_copy(
                table_hbm.at[idx_v.at[t]], bufs[b], gsems.at[b]).wait()
            _scale_block(bufs[b])
            pltpu.async_copy(
                bufs[b], out_hbm.at[t, pl.ds(col0, COLS_PER_W)], ssems.at[b])

    for t in (SEQ - 2, SEQ - 1):
        pltpu.make_async_copy(
            bufs[t % NBUF], out_hbm.at[0, pl.ds(col0, COLS_PER_W)],
            ssems.at[t % NBUF]).wait()


@functools.partial(jax.jit, static_argnames=())
def _emb_call(idx_t, table):
    mesh = plsc.VectorSubcoreMesh(core_axis_name="c", subcore_axis_name="s")
    k = pl.kernel(
        _emb_body,
        out_type=jax.ShapeDtypeStruct((SEQ, BATCH, D), jnp.float32),
        mesh=mesh,
        scratch_types=(
            [pltpu.VMEM((SEQ, COLS_PER_W), jnp.int32)]
            + [pltpu.VMEM((COLS_PER_W, D), jnp.float32) for _ in range(NBUF)]
            + [pltpu.SemaphoreType.DMA((NBUF,)),
               pltpu.SemaphoreType.DMA((NBUF,))]
        ),
    )
    return k(table, idx_t)


def kernel(x, table):
    idx_t = x.astype(jnp.int32).T
    out_t = _emb_call(idx_t, table)
    return out_t.transpose(1, 0, 2)


# speedup vs baseline: 9.4407x; 1.0024x over previous
import functools
import math

import jax
import jax.numpy as jnp
from jax import lax
from jax.experimental import pallas as pl
from jax.experimental.pallas import tpu as pltpu
from jax.experimental.pallas import tpu_sc as plsc

BATCH = 4096
SEQ = 50
D = 128
NC = 2
NS = 16
NW = NC * NS
COLS_PER_W = BATCH // NW
SCALE = math.sqrt(128.0)
L = 16
NBUF = 5


def _scale_block(buf):
    @pl.loop(0, COLS_PER_W, unroll=1)
    def _row(r):
        for seg in range(D // L):
            sl = pl.ds(seg * L, L)
            buf[r, sl] = buf[r, sl] * SCALE


def _emb_body(table_hbm, idx_hbm, out_hbm, idx_v,
              b0, b1, b2, b3, b4, gsems, ssems):
    c = lax.axis_index("c")
    s = lax.axis_index("s")
    wid = s * NC + c
    bufs = (b0, b1, b2, b3, b4)
    col0 = wid * COLS_PER_W
    pltpu.sync_copy(idx_hbm.at[:, pl.ds(col0, COLS_PER_W)], idx_v)

    for b in range(NBUF - 2):
        pltpu.async_copy(table_hbm.at[idx_v.at[b]], bufs[b], gsems.at[b])

    @pl.loop(0, SEQ, step=NBUF)
    def _pass(j):
        for b in range(NBUF):
            t = j + b
            nb = (b + NBUF - 2) % NBUF

            @pl.when(t >= 2)
            def _drain():
                pltpu.make_async_copy(
                    bufs[nb], out_hbm.at[0, pl.ds(col0, COLS_PER_W)],
                    ssems.at[nb]).wait()

            @pl.when(t + NBUF - 2 < SEQ)
            def _lookahead():
                pltpu.async_copy(
                    table_hbm.at[idx_v.at[t + NBUF - 2]], bufs[nb],
                    gsems.at[nb])

            pltpu.make_async_copy(
                table_hbm.at[idx_v.at[t]], bufs[b], gsems.at[b]).wait()
            _scale_block(bufs[b])
            pltpu.async_copy(
                bufs[b], out_hbm.at[t, pl.ds(col0, COLS_PER_W)], ssems.at[b])

    for t in (SEQ - 2, SEQ - 1):
        pltpu.make_async_copy(
            bufs[t % NBUF], out_hbm.at[0, pl.ds(col0, COLS_PER_W)],
            ssems.at[t % NBUF]).wait()


@functools.partial(jax.jit, static_argnames=())
def _emb_call(idx_t, table):
    mesh = plsc.VectorSubcoreMesh(core_axis_name="c", subcore_axis_name="s")
    k = pl.kernel(
        _emb_body,
        out_type=jax.ShapeDtypeStruct((SEQ, BATCH, D), jnp.float32),
        mesh=mesh,
        scratch_types=(
            [pltpu.VMEM((SEQ, COLS_PER_W), jnp.int32)]
            + [pltpu.VMEM((COLS_PER_W, D), jnp.float32) for _ in range(NBUF)]
            + [pltpu.SemaphoreType.DMA((NBUF,)),
               pltpu.SemaphoreType.DMA((NBUF,))]
        ),
    )
    return k(table, idx_t)


def kernel(x, table):
    idx_t = x.astype(jnp.int32).T
    out_t = _emb_call(idx_t, table)
    return out_t.transpose(1, 0, 2)
